# baseline (speedup 1.0000x reference)
import jax
import jax.numpy as jnp
from jax.experimental import pallas as pl
from jax.experimental.pallas import tpu as pltpu

_VOCAB = 100000
_D = 128
_K = 65

_NB = 4
_SROWS = 200
_BROWS = _SROWS * 128
_TOT_SROWS = _NB * _SROWS
_GR = 40
_NGRP = _TOT_SROWS // _GR
_GPB = _SROWS // _GR

_NEG = float("-inf")


def _body(wid_ref, wv_blk_ref, emb_blk_ref, out_s_ref, out_i_ref,
          scores_scr, gm_scr, gm2_scr, sel_scr):
    i = pl.program_id(0)

    wv = wv_blk_ref[pl.ds(wid_ref[0] % 8, 1), :]

    wv8 = jnp.broadcast_to(wv.astype(jnp.bfloat16).astype(jnp.float32),
                           (8, 128))
    scores = jax.lax.dot_general(
        emb_blk_ref[...].astype(jnp.bfloat16).astype(jnp.float32),
        wv8,
        dimension_numbers=(((1,), (1,)), ((), ())),
        preferred_element_type=jnp.float32,
    )
    s2 = scores[:, :1].reshape(_SROWS, 128)

    r_io = jax.lax.broadcasted_iota(jnp.int32, (_SROWS, 128), 0)
    c_io = jax.lax.broadcasted_iota(jnp.int32, (_SROWS, 128), 1)
    gidx = i * _BROWS + r_io * 128 + c_io
    s2 = jnp.where(gidx < _VOCAB, s2, _NEG)
    scores_scr[pl.ds(i * _SROWS, _SROWS), :] = s2

    gm_blk = jnp.max(s2.reshape(_GPB, _GR, 128), axis=1)
    gm_scr[pl.ds(i * _GPB, _GPB), :] = gm_blk

    bm = jnp.max(gm_blk, axis=0, keepdims=True)

    @pl.when(i == 0)
    def _():
        gm2_scr[...] = bm

    @pl.when(i > 0)
    def _():
        gm2_scr[...] = jnp.maximum(gm2_scr[...], bm)

    @pl.when(i == _NB - 1)
    def _():
        lane1 = jax.lax.broadcasted_iota(jnp.int32, (1, 128), 1)
        g_io = jax.lax.broadcasted_iota(jnp.int32, (_NGRP, 128), 0)
        laneg = jax.lax.broadcasted_iota(jnp.int32, (_NGRP, 128), 1)
        rG = jax.lax.broadcasted_iota(jnp.int32, (_GR, 128), 0)
        cG = jax.lax.broadcasted_iota(jnp.int32, (_GR, 128), 1)
        lin_f = jax.lax.broadcasted_iota(jnp.int32, (_TOT_SROWS, 128), 0) * 128 \
            + jax.lax.broadcasted_iota(jnp.int32, (_TOT_SROWS, 128), 1)
        k_io = jax.lax.broadcasted_iota(jnp.int32, (_K,), 0)
        big = jnp.int32(2**30)

        def step(k, _):
            gm2 = gm2_scr[...]
            m = jnp.max(gm2)
            hits2 = gm2 == m
            nc = jnp.sum(jnp.where(hits2, 1, 0))

            @pl.when(nc == 1)
            def _():
                c = jnp.min(jnp.where(hits2, lane1, big))
                gmv = gm_scr[...]
                g = jnp.min(jnp.where((gmv == m) & (laneg == c), g_io, big))
                rows = scores_scr[pl.ds(g * _GR, _GR), :]
                r = jnp.min(jnp.where((rows == m) & (cG == c), rG, big))
                sel_scr[0, 0] = g
                sel_scr[0, 1] = c
                sel_scr[0, 2] = r

            @pl.when(nc > 1)
            def _():
                s = scores_scr[...]
                eg = jnp.min(jnp.where(s == m, lin_f, big))
                row = eg // 128
                sel_scr[0, 0] = row // _GR
                sel_scr[0, 1] = eg - row * 128
                sel_scr[0, 2] = row - (row // _GR) * _GR

            g = sel_scr[0, 0]
            c = sel_scr[0, 1]
            r = sel_scr[0, 2]
            gid = (g * _GR + r) * 128 + c

            out_s_ref[...] = jnp.where(k_io == k, m, out_s_ref[...])
            out_i_ref[...] = jnp.where(k_io == k, gid, out_i_ref[...])

            rows = scores_scr[pl.ds(g * _GR, _GR), :]
            rows = jnp.where((rG == r) & (cG == c), _NEG, rows)
            scores_scr[pl.ds(g * _GR, _GR), :] = rows
            newv = jnp.max(jnp.where(cG == c, rows, _NEG))
            gmrow = gm_scr[pl.ds(g, 1), :]
            gmrow = jnp.where(lane1 == c, newv, gmrow)
            gm_scr[pl.ds(g, 1), :] = gmrow
            gmv = gm_scr[...]
            nv2 = jnp.max(jnp.where(laneg == c, gmv, _NEG))
            gm2_scr[...] = jnp.where(lane1 == c, nv2, gm2_scr[...])
            return 0

        jax.lax.fori_loop(0, _K, step, 0)


@jax.jit
def kernel(wordid, emb):
    wid = wordid.astype(jnp.int32)
    grid_spec = pltpu.PrefetchScalarGridSpec(
        num_scalar_prefetch=1,
        grid=(_NB,),
        in_specs=[
            pl.BlockSpec((8, 128), lambda i, w: (w[0] // 8, 0)),
            pl.BlockSpec((_BROWS, 128), lambda i, w: (i, 0)),
        ],
        out_specs=[
            pl.BlockSpec((_K,), lambda i, w: (0,)),
            pl.BlockSpec((_K,), lambda i, w: (0,)),
        ],
        scratch_shapes=[
            pltpu.VMEM((_TOT_SROWS, 128), jnp.float32),
            pltpu.VMEM((_NGRP, 128), jnp.float32),
            pltpu.VMEM((1, 128), jnp.float32),
            pltpu.SMEM((1, 3), jnp.int32),
        ],
    )
    scores, ids = pl.pallas_call(
        _body,
        grid_spec=grid_spec,
        out_shape=[
            jax.ShapeDtypeStruct((_K,), jnp.float32),
            jax.ShapeDtypeStruct((_K,), jnp.int32),
        ],
    )(wid, emb, emb)
    return scores, ids


# speedup vs baseline: 1.4540x; 1.4540x over previous
import jax
import jax.numpy as jnp
from jax.experimental import pallas as pl
from jax.experimental.pallas import tpu as pltpu

_VOCAB = 100000
_D = 128
_K = 65

_NB = 4
_SROWS = 200
_BROWS = _SROWS * 128
_TOT_SROWS = _NB * _SROWS
_GR = 40
_NGRP = _TOT_SROWS // _GR
_GPB = _SROWS // _GR

_NEG = float("-inf")


def _body(wid_ref, wv_blk_ref, emb_blk_ref, out_s_ref, out_i_ref,
          scores_scr, gm_scr):
    i = pl.program_id(0)

    wv = wv_blk_ref[pl.ds(wid_ref[0] % 8, 1), :]

    wv8 = jnp.broadcast_to(wv.astype(jnp.bfloat16).astype(jnp.float32),
                           (8, 128))
    scores = jax.lax.dot_general(
        emb_blk_ref[...].astype(jnp.bfloat16).astype(jnp.float32),
        wv8,
        dimension_numbers=(((1,), (1,)), ((), ())),
        preferred_element_type=jnp.float32,
    )
    s2 = scores[:, :1].reshape(_SROWS, 128)

    r_io = jax.lax.broadcasted_iota(jnp.int32, (_SROWS, 128), 0)
    c_io = jax.lax.broadcasted_iota(jnp.int32, (_SROWS, 128), 1)
    gidx = i * _BROWS + r_io * 128 + c_io
    s2 = jnp.where(gidx < _VOCAB, s2, _NEG)
    scores_scr[pl.ds(i * _SROWS, _SROWS), :] = s2

    gm_blk = jnp.max(s2.reshape(_GPB, _GR, 128), axis=1)
    gm_scr[pl.ds(i * _GPB, _GPB), :] = gm_blk

    @pl.when(i == _NB - 1)
    def _():
        g_io = jax.lax.broadcasted_iota(jnp.int32, (_NGRP, 128), 0)
        keyG = jax.lax.broadcasted_iota(jnp.int32, (_GR, 128), 0) * 128 + \
            jax.lax.broadcasted_iota(jnp.int32, (_GR, 128), 1)
        lane1 = jax.lax.broadcasted_iota(jnp.int32, (1, 128), 1)
        k_io = jax.lax.broadcasted_iota(jnp.int32, (_K,), 0)
        big = jnp.int32(2**30)

        def step(k, _):
            gmv = gm_scr[...]
            m_b = jnp.max(gmv, axis=(0, 1), keepdims=True)
            g = jnp.min(jnp.where(gmv == m_b, g_io, big))

            rows = scores_scr[pl.ds(g * _GR, _GR), :]
            key_b = jnp.min(jnp.where(rows == m_b, keyG, big),
                            axis=(0, 1), keepdims=True)
            c_b = jax.lax.rem(key_b, jnp.int32(128))
            gid_b = g * (_GR * 128) + key_b

            out_s_ref[...] = jnp.where(
                k_io == k, jnp.broadcast_to(m_b.reshape(1), (_K,)),
                out_s_ref[...])
            out_i_ref[...] = jnp.where(
                k_io == k, jnp.broadcast_to(gid_b.reshape(1), (_K,)),
                out_i_ref[...])

            rows = jnp.where(keyG == key_b, _NEG, rows)
            scores_scr[pl.ds(g * _GR, _GR), :] = rows
            newv_b = jnp.max(jnp.where((keyG % 128) == c_b, rows, _NEG),
                             axis=(0, 1), keepdims=True)
            gmrow = gm_scr[pl.ds(g, 1), :]
            gm_scr[pl.ds(g, 1), :] = jnp.where(
                lane1 == c_b, jnp.broadcast_to(newv_b, (1, 128)), gmrow)
            return 0

        jax.lax.fori_loop(0, _K, step, 0)


@jax.jit
def kernel(wordid, emb):
    wid = wordid.astype(jnp.int32)
    grid_spec = pltpu.PrefetchScalarGridSpec(
        num_scalar_prefetch=1,
        grid=(_NB,),
        in_specs=[
            pl.BlockSpec((8, 128), lambda i, w: (w[0] // 8, 0)),
            pl.BlockSpec((_BROWS, 128), lambda i, w: (i, 0)),
        ],
        out_specs=[
            pl.BlockSpec((_K,), lambda i, w: (0,)),
            pl.BlockSpec((_K,), lambda i, w: (0,)),
        ],
        scratch_shapes=[
            pltpu.VMEM((_TOT_SROWS, 128), jnp.float32),
            pltpu.VMEM((_NGRP, 128), jnp.float32),
        ],
    )
    scores, ids = pl.pallas_call(
        _body,
        grid_spec=grid_spec,
        out_shape=[
            jax.ShapeDtypeStruct((_K,), jnp.float32),
            jax.ShapeDtypeStruct((_K,), jnp.int32),
        ],
    )(wid, emb, emb)
    return scores, ids


# speedup vs baseline: 1.6476x; 1.1331x over previous
import jax
import jax.numpy as jnp
from jax.experimental import pallas as pl
from jax.experimental.pallas import tpu as pltpu

_VOCAB = 100000
_D = 128
_K = 65

_NB = 4
_SROWS = 200
_BROWS = _SROWS * 128
_TOT_SROWS = _NB * _SROWS
_NGRP = _TOT_SROWS // 8
_GPB = _SROWS // 8

_NEG = float("-inf")


def _body(wid_ref, wv_blk_ref, emb_blk_ref, out_s_ref, out_i_ref,
          scores_scr, gm_scr):
    i = pl.program_id(0)

    wv = wv_blk_ref[pl.ds(wid_ref[0] % 8, 1), :]

    scores = jax.lax.dot_general(
        emb_blk_ref[...].astype(jnp.bfloat16).astype(jnp.float32),
        wv.astype(jnp.bfloat16).astype(jnp.float32),
        dimension_numbers=(((1,), (1,)), ((), ())),
        preferred_element_type=jnp.float32,
    )
    s2 = scores.reshape(_SROWS, 128)

    r_io = jax.lax.broadcasted_iota(jnp.int32, (_SROWS, 128), 0)
    c_io = jax.lax.broadcasted_iota(jnp.int32, (_SROWS, 128), 1)
    gidx = i * _BROWS + r_io * 128 + c_io
    s2 = jnp.where(gidx < _VOCAB, s2, _NEG)
    scores_scr[pl.ds(i * _SROWS, _SROWS), :] = s2

    gm_blk = jnp.max(s2.reshape(_GPB, 8, 128), axis=1)
    gm_scr[pl.ds(i * _GPB, _GPB), :] = gm_blk

    @pl.when(i == _NB - 1)
    def _():
        lin_g = jax.lax.broadcasted_iota(jnp.int32, (_NGRP, 128), 0) * 128 + \
                jax.lax.broadcasted_iota(jnp.int32, (_NGRP, 128), 1)
        r8 = jax.lax.broadcasted_iota(jnp.int32, (8, 128), 0)
        c8 = jax.lax.broadcasted_iota(jnp.int32, (8, 128), 1)
        k_io = jax.lax.broadcasted_iota(jnp.int32, (_K,), 0)
        big = jnp.int32(2**30)

        def step(k, _):
            g = gm_scr[...]
            m = jnp.max(g)
            eg = jnp.min(jnp.where(g == m, lin_g, big))
            grp = eg // 128
            c = eg - grp * 128

            rows = scores_scr[pl.ds(grp * 8, 8), :]
            hit = (rows == m) & (c8 == c)
            r = jnp.min(jnp.where(hit, r8, big))
            gid = (grp * 8 + r) * 128 + c

            out_s_ref[...] = jnp.where(k_io == k, m, out_s_ref[...])
            out_i_ref[...] = jnp.where(k_io == k, gid, out_i_ref[...])

            rows = jnp.where((r8 == r) & (c8 == c), _NEG, rows)
            scores_scr[pl.ds(grp * 8, 8), :] = rows
            gm_scr[pl.ds(grp, 1), :] = jnp.max(rows, axis=0, keepdims=True)
            return 0

        jax.lax.fori_loop(0, _K, step, 0)


@jax.jit
def kernel(wordid, emb):
    wid = wordid.astype(jnp.int32)
    grid_spec = pltpu.PrefetchScalarGridSpec(
        num_scalar_prefetch=1,
        grid=(_NB,),
        in_specs=[
            pl.BlockSpec((8, 128), lambda i, w: (w[0] // 8, 0)),
            pl.BlockSpec((_BROWS, 128), lambda i, w: (i, 0)),
        ],
        out_specs=[
            pl.BlockSpec((_K,), lambda i, w: (0,)),
            pl.BlockSpec((_K,), lambda i, w: (0,)),
        ],
        scratch_shapes=[
            pltpu.VMEM((_TOT_SROWS, 128), jnp.float32),
            pltpu.VMEM((_NGRP, 128), jnp.float32),
        ],
    )
    scores, ids = pl.pallas_call(
        _body,
        grid_spec=grid_spec,
        out_shape=[
            jax.ShapeDtypeStruct((_K,), jnp.float32),
            jax.ShapeDtypeStruct((_K,), jnp.int32),
        ],
    )(wid, emb, emb)
    return scores, ids
